# Initial kernel scaffold; baseline (speedup 1.0000x reference)
import functools

import jax
import jax.numpy as jnp
from jax import lax
from jax.experimental import pallas as pl
from jax.experimental.pallas import tpu as pltpu
from jax.experimental.pallas import tpu_sc as plsc

NW = 32
B = 128


def _tc_h0(x, W1, NP, BN):
    N, F = x.shape
    H = W1.shape[1]

    def body(x_ref, w_ref, o_ref):
        o_ref[...] = jnp.dot(x_ref[...], w_ref[...],
                             preferred_element_type=jnp.float32)

    return pl.pallas_call(
        body,
        grid=(N // BN,),
        in_specs=[pl.BlockSpec((BN, F), lambda i: (i, 0)),
                  pl.BlockSpec((F, H), lambda i: (0, 0))],
        out_specs=pl.BlockSpec((BN, H), lambda i: (i, 0)),
        out_shape=jax.ShapeDtypeStruct((NP, H), jnp.float32),
    )(x, W1)


def _sc_deg(dst2, z1, ones, NP):
    R = dst2.shape[0] // NW
    NS = NP // 16
    mesh = plsc.VectorSubcoreMesh(core_axis_name="c", subcore_axis_name="s")

    @functools.partial(
        pl.kernel,
        out_type=jax.ShapeDtypeStruct((2 * NP,), jnp.float32),
        mesh=mesh,
        scratch_types=[
            pltpu.VMEM((R, B), jnp.int32),
            pltpu.VMEM((B,), jnp.float32),
            pltpu.VMEM_SHARED((NP,), jnp.float32),
        ],
    )
    def k(dst_hbm, z1_hbm, ones_hbm, deg_hbm, idx_v, ones_v, deg_sp):
        c = lax.axis_index("c")
        s = lax.axis_index("s")
        w = s * 2 + c
        lo = s * NS
        pltpu.sync_copy(z1_hbm.at[pl.ds(lo, NS)], deg_sp.at[pl.ds(lo, NS)])
        pltpu.sync_copy(ones_hbm, ones_v)
        pltpu.sync_copy(dst_hbm.at[pl.ds(w * R, R)], idx_v)
        plsc.subcore_barrier()

        @pl.loop(0, R)
        def _(j):
            pltpu.sync_copy(ones_v, deg_sp.at[idx_v.at[j]], add=True)

        plsc.subcore_barrier()
        pltpu.sync_copy(deg_sp.at[pl.ds(lo, NS)],
                        deg_hbm.at[pl.ds(c * NP + lo, NS)])

    return k(dst2, z1, ones)


def _tc_dinv_g0(deg2, h0, N, NP, BN):
    H = h0.shape[1]

    def body(d_ref, h_ref, dinv_ref, g0_ref):
        deg = d_ref[0] + d_ref[1] + 1.0
        dinv = lax.rsqrt(deg)
        dinv_ref[...] = dinv
        g0_ref[...] = h_ref[...] * dinv

    return pl.pallas_call(
        body,
        grid=(N // BN,),
        in_specs=[pl.BlockSpec((2, BN, 1), lambda i: (0, i, 0)),
                  pl.BlockSpec((BN, H), lambda i: (i, 0))],
        out_specs=[pl.BlockSpec((BN, 1), lambda i: (i, 0)),
                   pl.BlockSpec((BN, H), lambda i: (i, 0))],
        out_shape=[jax.ShapeDtypeStruct((NP, 1), jnp.float32),
                   jax.ShapeDtypeStruct((NP, H), jnp.float32)],
    )(deg2.reshape(2, NP, 1), h0)


def _sc_agg(src2, dst2, g0, dinv, z1, z2, NP):
    R = src2.shape[0] // NW
    H = g0.shape[1]
    NS = NP // 16
    mesh = plsc.VectorSubcoreMesh(core_axis_name="c", subcore_axis_name="s")

    @functools.partial(
        pl.kernel,
        out_type=(jax.ShapeDtypeStruct((2 * NP, H), jnp.float32),
                  jax.ShapeDtypeStruct((2 * NP,), jnp.float32)),
        mesh=mesh,
        scratch_types=[
            pltpu.VMEM((R, B), jnp.int32),
            pltpu.VMEM((R, B), jnp.int32),
            pltpu.VMEM((B, H), jnp.float32),
            pltpu.VMEM((B,), jnp.float32),
            pltpu.VMEM_SHARED((NP, H), jnp.float32),
            pltpu.VMEM_SHARED((NP,), jnp.float32),
            pltpu.VMEM_SHARED((NP,), jnp.float32),
            pltpu.SemaphoreType.DMA,
            pltpu.SemaphoreType.DMA,
        ],
    )
    def k(src_hbm, dst_hbm, g0_hbm, dinv_hbm, z1_hbm, z2_hbm,
          acc_out, t_out, sidx, didx, rows, dvals,
          acc_sp, t_sp, dinv_sp, gsem, dsem):
        c = lax.axis_index("c")
        s = lax.axis_index("s")
        w = s * 2 + c
        lo = s * NS
        pltpu.sync_copy(z2_hbm.at[pl.ds(lo, NS)], acc_sp.at[pl.ds(lo, NS)])
        pltpu.sync_copy(z1_hbm.at[pl.ds(lo, NS)], t_sp.at[pl.ds(lo, NS)])
        pltpu.sync_copy(dinv_hbm.at[pl.ds(lo, NS)], dinv_sp.at[pl.ds(lo, NS)])
        pltpu.sync_copy(src_hbm.at[pl.ds(w * R, R)], sidx)
        pltpu.sync_copy(dst_hbm.at[pl.ds(w * R, R)], didx)
        plsc.subcore_barrier()

        @pl.loop(0, R)
        def _(j):
            pltpu.async_copy(g0_hbm.at[sidx.at[j]], rows, gsem).wait()
            pltpu.sync_copy(rows, acc_sp.at[didx.at[j]], add=True)
            pltpu.async_copy(dinv_sp.at[didx.at[j]], dvals, dsem).wait()
            pltpu.sync_copy(dvals, t_sp.at[sidx.at[j]], add=True)

        plsc.subcore_barrier()
        pltpu.sync_copy(acc_sp.at[pl.ds(lo, NS)],
                        acc_out.at[pl.ds(c * NP + lo, NS)])
        pltpu.sync_copy(t_sp.at[pl.ds(lo, NS)],
                        t_out.at[pl.ds(c * NP + lo, NS)])

    return k(src2, dst2, g0, dinv, z1, z2)


def _tc_final(acc2, t2, dinv, g0, b1, W2, b2, N, NP, BN):
    H = g0.shape[1]
    M = W2.shape[1]
    nblk = N // BN

    def body(a_ref, t_ref, dv_ref, g0_ref, b1_ref, w2_ref, b2_ref,
             o_ref, pool_ref):
        i = pl.program_id(0)
        accs = a_ref[0] + a_ref[1]
        ts = t_ref[0] + t_ref[1]
        dv = dv_ref[...]
        h1 = jnp.maximum(dv * (accs + g0_ref[...]) + b1_ref[...], 0.0)
        cvec = dv * (ts + dv)
        part = jnp.sum(cvec * h1, axis=0, keepdims=True)

        @pl.when(i == 0)
        def _():
            pool_ref[...] = jnp.zeros_like(pool_ref)

        pool_ref[...] += part

        @pl.when(i == nblk - 1)
        def _():
            o_ref[...] = jnp.dot(pool_ref[...] * (1.0 / N), w2_ref[...],
                                 preferred_element_type=jnp.float32) \
                         + b2_ref[...]

    return pl.pallas_call(
        body,
        grid=(nblk,),
        in_specs=[pl.BlockSpec((2, BN, H), lambda i: (0, i, 0)),
                  pl.BlockSpec((2, BN, 1), lambda i: (0, i, 0)),
                  pl.BlockSpec((BN, 1), lambda i: (i, 0)),
                  pl.BlockSpec((BN, H), lambda i: (i, 0)),
                  pl.BlockSpec((1, H), lambda i: (0, 0)),
                  pl.BlockSpec((H, M), lambda i: (0, 0)),
                  pl.BlockSpec((1, M), lambda i: (0, 0))],
        out_specs=pl.BlockSpec((1, M), lambda i: (0, 0)),
        out_shape=jax.ShapeDtypeStruct((1, M), jnp.float32),
        scratch_shapes=[pltpu.VMEM((1, H), jnp.float32)],
    )(acc2.reshape(2, NP, H), t2.reshape(2, NP, 1), dinv, g0,
      b1.reshape(1, H), W2, b2.reshape(1, M))


def kernel(x, edge_index, W1, b1, W2, b2):
    N, F = x.shape
    H = W1.shape[1]
    E = edge_index.shape[1]

    NP = ((N + 127) // 128) * 128
    R = -(-E // (NW * B))
    EP = NW * R * B
    pad = EP - E

    padidx = N + (jnp.arange(pad, dtype=jnp.int32) % 16)
    src2 = jnp.concatenate([edge_index[0], padidx]).reshape(NW * R, B)
    dst2 = jnp.concatenate([edge_index[1], padidx]).reshape(NW * R, B)

    z1 = jnp.zeros((NP,), jnp.float32)
    z2 = jnp.zeros((NP, H), jnp.float32)
    ones = jnp.ones((B,), jnp.float32)

    h0 = _tc_h0(x, W1, NP, 1000)
    deg2 = _sc_deg(dst2, z1, ones, NP)
    dinv, g0 = _tc_dinv_g0(deg2, h0, N, NP, 1000)
    acc2, t2 = _sc_agg(src2, dst2, g0, dinv.reshape(NP), z1, z2, NP)
    return _tc_final(acc2, t2, dinv, g0, b1, W2, b2, N, NP, 1000)


# speedup vs baseline: 34.4427x; 34.4427x over previous
import functools

import jax
import jax.numpy as jnp
from jax import lax
from jax.experimental import pallas as pl
from jax.experimental.pallas import tpu as pltpu
from jax.experimental.pallas import tpu_sc as plsc

NW = 32
B = 128


def _tc_h0(x, W1, NP, BN):
    N, F = x.shape
    H = W1.shape[1]

    def body(x_ref, w_ref, o_ref):
        o_ref[...] = jnp.dot(x_ref[...], w_ref[...],
                             preferred_element_type=jnp.float32)

    return pl.pallas_call(
        body,
        grid=(N // BN,),
        in_specs=[pl.BlockSpec((BN, F), lambda i: (i, 0)),
                  pl.BlockSpec((F, H), lambda i: (0, 0))],
        out_specs=pl.BlockSpec((BN, H), lambda i: (i, 0)),
        out_shape=jax.ShapeDtypeStruct((NP, H), jnp.float32),
    )(x, W1)


def _sc_deg(dst2, z1, ones, NP):
    R = dst2.shape[0] // NW
    NS = NP // 16
    mesh = plsc.VectorSubcoreMesh(core_axis_name="c", subcore_axis_name="s")

    @functools.partial(
        pl.kernel,
        out_type=jax.ShapeDtypeStruct((2 * NP,), jnp.float32),
        mesh=mesh,
        scratch_types=[
            pltpu.VMEM((R, B), jnp.int32),
            pltpu.VMEM((B,), jnp.float32),
            pltpu.VMEM_SHARED((NP,), jnp.float32),
        ],
        compiler_params=pltpu.CompilerParams(use_tc_tiling_on_sc=False),
    )
    def k(dst_hbm, z1_hbm, ones_hbm, deg_hbm, idx_v, ones_v, deg_sp):
        c = lax.axis_index("c")
        s = lax.axis_index("s")
        w = s * 2 + c
        lo = s * NS
        pltpu.sync_copy(z1_hbm.at[pl.ds(lo, NS)], deg_sp.at[pl.ds(lo, NS)])
        pltpu.sync_copy(ones_hbm, ones_v)
        pltpu.sync_copy(dst_hbm.at[pl.ds(w * R, R)], idx_v)
        plsc.subcore_barrier()

        @pl.loop(0, R)
        def _(j):
            pltpu.sync_copy(ones_v, deg_sp.at[idx_v.at[j]], add=True)

        plsc.subcore_barrier()
        pltpu.sync_copy(deg_sp.at[pl.ds(lo, NS)],
                        deg_hbm.at[pl.ds(c * NP + lo, NS)])

    return k(dst2, z1, ones)


def _tc_dinv_g0(deg2, h0, N, NP, BN):
    H = h0.shape[1]

    def body(d_ref, h_ref, dinv_ref, g0_ref):
        deg = d_ref[0] + d_ref[1] + 1.0
        dinv = lax.rsqrt(deg)
        dinv_ref[...] = dinv
        g0_ref[...] = h_ref[...] * dinv

    return pl.pallas_call(
        body,
        grid=(N // BN,),
        in_specs=[pl.BlockSpec((2, BN, 1), lambda i: (0, i, 0)),
                  pl.BlockSpec((BN, H), lambda i: (i, 0))],
        out_specs=[pl.BlockSpec((BN, 1), lambda i: (i, 0)),
                   pl.BlockSpec((BN, H), lambda i: (i, 0))],
        out_shape=[jax.ShapeDtypeStruct((NP, 1), jnp.float32),
                   jax.ShapeDtypeStruct((NP, H), jnp.float32)],
    )(deg2.reshape(2, NP, 1), h0)


def _sc_agg(src2, dst2, g0, dinv, z1, z2, NP):
    R = src2.shape[0] // NW
    H = g0.shape[1]
    NS = NP // 16
    C = next(c for c in range(min(32, R), 0, -1) if R % c == 0)
    NC = R // C
    mesh = plsc.VectorSubcoreMesh(core_axis_name="c", subcore_axis_name="s")

    @functools.partial(
        pl.kernel,
        out_type=(jax.ShapeDtypeStruct((2 * NP, H), jnp.float32),
                  jax.ShapeDtypeStruct((2 * NP,), jnp.float32)),
        mesh=mesh,
        scratch_types=[
            pltpu.VMEM((C, B), jnp.int32),
            pltpu.VMEM((C, B), jnp.int32),
            pltpu.VMEM((B, H), jnp.float32),
            pltpu.VMEM((B,), jnp.float32),
            pltpu.VMEM_SHARED((NP, H), jnp.float32),
            pltpu.VMEM_SHARED((NP,), jnp.float32),
            pltpu.VMEM_SHARED((NP,), jnp.float32),
            pltpu.SemaphoreType.DMA,
            pltpu.SemaphoreType.DMA,
        ],
        compiler_params=pltpu.CompilerParams(use_tc_tiling_on_sc=False),
    )
    def k(src_hbm, dst_hbm, g0_hbm, dinv_hbm, z1_hbm, z2_hbm,
          acc_out, t_out, sidx, didx, rows, dvals,
          acc_sp, t_sp, dinv_sp, gsem, dsem):
        c = lax.axis_index("c")
        s = lax.axis_index("s")
        w = s * 2 + c
        lo = s * NS
        pltpu.sync_copy(z2_hbm.at[pl.ds(lo, NS)], acc_sp.at[pl.ds(lo, NS)])
        pltpu.sync_copy(z1_hbm.at[pl.ds(lo, NS)], t_sp.at[pl.ds(lo, NS)])
        pltpu.sync_copy(dinv_hbm.at[pl.ds(lo, NS)], dinv_sp.at[pl.ds(lo, NS)])
        plsc.subcore_barrier()

        @pl.loop(0, NC)
        def _(o):
            pltpu.sync_copy(src_hbm.at[pl.ds(w * R + o * C, C)], sidx)
            pltpu.sync_copy(dst_hbm.at[pl.ds(w * R + o * C, C)], didx)

            @pl.loop(0, C)
            def _(j):
                pltpu.async_copy(g0_hbm.at[sidx.at[j]], rows, gsem).wait()
                pltpu.sync_copy(rows, acc_sp.at[didx.at[j]], add=True)
                pltpu.async_copy(dinv_sp.at[didx.at[j]], dvals, dsem).wait()
                pltpu.sync_copy(dvals, t_sp.at[sidx.at[j]], add=True)

        plsc.subcore_barrier()
        pltpu.sync_copy(acc_sp.at[pl.ds(lo, NS)],
                        acc_out.at[pl.ds(c * NP + lo, NS)])
        pltpu.sync_copy(t_sp.at[pl.ds(lo, NS)],
                        t_out.at[pl.ds(c * NP + lo, NS)])

    return k(src2, dst2, g0, dinv, z1, z2)


def _tc_final(acc2, t2, dinv, g0, b1, W2, b2, N, NP, BN):
    H = g0.shape[1]
    M = W2.shape[1]
    nblk = N // BN

    def body(a_ref, t_ref, dv_ref, g0_ref, b1_ref, w2_ref, b2_ref,
             o_ref, pool_ref):
        i = pl.program_id(0)
        accs = a_ref[0] + a_ref[1]
        ts = t_ref[0] + t_ref[1]
        dv = dv_ref[...]
        h1 = jnp.maximum(dv * (accs + g0_ref[...]) + b1_ref[...], 0.0)
        cvec = dv * (ts + dv)
        part = jnp.sum(cvec * h1, axis=0, keepdims=True)

        @pl.when(i == 0)
        def _():
            pool_ref[...] = jnp.zeros_like(pool_ref)

        pool_ref[...] += part

        @pl.when(i == nblk - 1)
        def _():
            o_ref[...] = jnp.dot(pool_ref[...] * (1.0 / N), w2_ref[...],
                                 preferred_element_type=jnp.float32) \
                         + b2_ref[...]

    return pl.pallas_call(
        body,
        grid=(nblk,),
        in_specs=[pl.BlockSpec((2, BN, H), lambda i: (0, i, 0)),
                  pl.BlockSpec((2, BN, 1), lambda i: (0, i, 0)),
                  pl.BlockSpec((BN, 1), lambda i: (i, 0)),
                  pl.BlockSpec((BN, H), lambda i: (i, 0)),
                  pl.BlockSpec((1, H), lambda i: (0, 0)),
                  pl.BlockSpec((H, M), lambda i: (0, 0)),
                  pl.BlockSpec((1, M), lambda i: (0, 0))],
        out_specs=pl.BlockSpec((1, M), lambda i: (0, 0)),
        out_shape=jax.ShapeDtypeStruct((1, M), jnp.float32),
        scratch_shapes=[pltpu.VMEM((1, H), jnp.float32)],
    )(acc2.reshape(2, NP, H), t2.reshape(2, NP, 1), dinv, g0,
      b1.reshape(1, H), W2, b2.reshape(1, M))


def kernel(x, edge_index, W1, b1, W2, b2):
    N, F = x.shape
    H = W1.shape[1]
    E = edge_index.shape[1]

    NP = ((N + 127) // 128) * 128
    R = -(-E // (NW * B))
    EP = NW * R * B
    pad = EP - E

    padidx = N + (jnp.arange(pad, dtype=jnp.int32) % 16)
    src2 = jnp.concatenate([edge_index[0], padidx]).reshape(NW * R, B)
    dst2 = jnp.concatenate([edge_index[1], padidx]).reshape(NW * R, B)

    z1 = jnp.zeros((NP,), jnp.float32)
    z2 = jnp.zeros((NP, H), jnp.float32)
    ones = jnp.ones((B,), jnp.float32)

    h0 = _tc_h0(x, W1, NP, 1000)
    deg2 = _sc_deg(dst2, z1, ones, NP)
    dinv, g0 = _tc_dinv_g0(deg2, h0, N, NP, 1000)
    acc2, t2 = _sc_agg(src2, dst2, g0, dinv.reshape(NP), z1, z2, NP)
    return _tc_final(acc2, t2, dinv, g0, b1, W2, b2, N, NP, 1000)


# speedup vs baseline: 40.0363x; 1.1624x over previous
import functools

import jax
import jax.numpy as jnp
from jax import lax
from jax.experimental import pallas as pl
from jax.experimental.pallas import tpu as pltpu
from jax.experimental.pallas import tpu_sc as plsc

NW = 32
B = 128


def _tc_h0(x, W1, NP, BN):
    N, F = x.shape
    H = W1.shape[1]

    def body(x_ref, w_ref, o_ref):
        o_ref[...] = jnp.dot(x_ref[...], w_ref[...],
                             preferred_element_type=jnp.float32)

    return pl.pallas_call(
        body,
        grid=(N // BN,),
        in_specs=[pl.BlockSpec((BN, F), lambda i: (i, 0)),
                  pl.BlockSpec((F, H), lambda i: (0, 0))],
        out_specs=pl.BlockSpec((BN, H), lambda i: (i, 0)),
        out_shape=jax.ShapeDtypeStruct((NP, H), jnp.float32),
    )(x, W1)


def _sc_deg(dst2, z1, ones, NP):
    R = dst2.shape[0] // NW
    NS = NP // 16
    mesh = plsc.VectorSubcoreMesh(core_axis_name="c", subcore_axis_name="s")

    @functools.partial(
        pl.kernel,
        out_type=jax.ShapeDtypeStruct((2 * NP,), jnp.float32),
        mesh=mesh,
        scratch_types=[
            pltpu.VMEM((R, B), jnp.int32),
            pltpu.VMEM((B,), jnp.float32),
            pltpu.VMEM_SHARED((NP,), jnp.float32),
        ],
        compiler_params=pltpu.CompilerParams(use_tc_tiling_on_sc=False),
    )
    def k(dst_hbm, z1_hbm, ones_hbm, deg_hbm, idx_v, ones_v, deg_sp):
        c = lax.axis_index("c")
        s = lax.axis_index("s")
        w = s * 2 + c
        lo = s * NS
        pltpu.sync_copy(z1_hbm.at[pl.ds(lo, NS)], deg_sp.at[pl.ds(lo, NS)])
        pltpu.sync_copy(ones_hbm, ones_v)
        pltpu.sync_copy(dst_hbm.at[pl.ds(w * R, R)], idx_v)
        plsc.subcore_barrier()

        @pl.loop(0, R)
        def _(j):
            pltpu.sync_copy(ones_v, deg_sp.at[idx_v.at[j]], add=True)

        plsc.subcore_barrier()
        pltpu.sync_copy(deg_sp.at[pl.ds(lo, NS)],
                        deg_hbm.at[pl.ds(c * NP + lo, NS)])

    return k(dst2, z1, ones)


def _tc_dinv_g0(deg2, h0, N, NP, BN):
    H = h0.shape[1]

    def body(d_ref, h_ref, dinv_ref, g0_ref):
        deg = d_ref[0] + d_ref[1] + 1.0
        dinv = lax.rsqrt(deg)
        dinv_ref[...] = dinv
        g0_ref[...] = h_ref[...] * dinv

    return pl.pallas_call(
        body,
        grid=(N // BN,),
        in_specs=[pl.BlockSpec((2, BN, 1), lambda i: (0, i, 0)),
                  pl.BlockSpec((BN, H), lambda i: (i, 0))],
        out_specs=[pl.BlockSpec((BN, 1), lambda i: (i, 0)),
                   pl.BlockSpec((BN, H), lambda i: (i, 0))],
        out_shape=[jax.ShapeDtypeStruct((NP, 1), jnp.float32),
                   jax.ShapeDtypeStruct((NP, H), jnp.float32)],
    )(deg2.reshape(2, NP, 1), h0)


NBUF = 4
CCH = 16


def _sc_agg(src2, dst2, g0, dinv, z1, z2, NP):
    R = src2.shape[0] // NW
    H = g0.shape[1]
    NS = NP // 16
    C = CCH
    NC = R // C
    assert R % C == 0 and C % NBUF == 0
    mesh = plsc.VectorSubcoreMesh(core_axis_name="c", subcore_axis_name="s")

    @functools.partial(
        pl.kernel,
        out_type=(jax.ShapeDtypeStruct((2 * NP, H), jnp.float32),
                  jax.ShapeDtypeStruct((2 * NP,), jnp.float32)),
        mesh=mesh,
        scratch_types=[
            pltpu.VMEM((2 * C, B), jnp.int32),
            pltpu.VMEM((2 * C, B), jnp.int32),
            pltpu.VMEM((NBUF, B, H), jnp.float32),
            pltpu.VMEM((NBUF, B), jnp.float32),
            pltpu.VMEM_SHARED((NP, H), jnp.float32),
            pltpu.VMEM_SHARED((NP,), jnp.float32),
            pltpu.VMEM_SHARED((NP,), jnp.float32),
        ] + [pltpu.SemaphoreType.DMA] * (2 * NBUF + 2),
        compiler_params=pltpu.CompilerParams(use_tc_tiling_on_sc=False),
    )
    def k(src_hbm, dst_hbm, g0_hbm, dinv_hbm, z1_hbm, z2_hbm,
          acc_out, t_out, sidx, didx, rows, dvals,
          acc_sp, t_sp, dinv_sp, *sems):
        gsem = sems[:NBUF]
        dsem = sems[NBUF:2 * NBUF]
        isem_s, isem_d = sems[2 * NBUF], sems[2 * NBUF + 1]
        c = lax.axis_index("c")
        s = lax.axis_index("s")
        w = s * 2 + c
        lo = s * NS
        pltpu.sync_copy(z2_hbm.at[pl.ds(lo, NS)], acc_sp.at[pl.ds(lo, NS)])
        pltpu.sync_copy(z1_hbm.at[pl.ds(lo, NS)], t_sp.at[pl.ds(lo, NS)])
        pltpu.sync_copy(dinv_hbm.at[pl.ds(lo, NS)], dinv_sp.at[pl.ds(lo, NS)])
        plsc.subcore_barrier()

        pltpu.async_copy(src_hbm.at[pl.ds(w * R, C)],
                         sidx.at[pl.ds(0, C)], isem_s)
        pltpu.async_copy(dst_hbm.at[pl.ds(w * R, C)],
                         didx.at[pl.ds(0, C)], isem_d)

        @pl.loop(0, NC)
        def _(o):
            base = lax.rem(o, 2) * C
            nbase = C - base
            pltpu.make_async_copy(src_hbm.at[pl.ds(0, C)],
                                  sidx.at[pl.ds(base, C)], isem_s).wait()
            pltpu.make_async_copy(dst_hbm.at[pl.ds(0, C)],
                                  didx.at[pl.ds(base, C)], isem_d).wait()

            @pl.when(o < NC - 1)
            def _():
                pltpu.async_copy(src_hbm.at[pl.ds(w * R + (o + 1) * C, C)],
                                 sidx.at[pl.ds(nbase, C)], isem_s)
                pltpu.async_copy(dst_hbm.at[pl.ds(w * R + (o + 1) * C, C)],
                                 didx.at[pl.ds(nbase, C)], isem_d)

            @pl.loop(0, C, step=NBUF)
            def _(g):
                gh, dh = [], []
                for b in range(NBUF):
                    j = base + g + b
                    gh.append(pltpu.async_copy(
                        g0_hbm.at[sidx.at[j]], rows.at[b], gsem[b]))
                    dh.append(pltpu.async_copy(
                        dinv_sp.at[didx.at[j]], dvals.at[b], dsem[b]))
                for b in range(NBUF):
                    j = base + g + b
                    gh[b].wait()
                    pltpu.sync_copy(rows.at[b], acc_sp.at[didx.at[j]],
                                    add=True)
                    dh[b].wait()
                    pltpu.sync_copy(dvals.at[b], t_sp.at[sidx.at[j]],
                                    add=True)

        plsc.subcore_barrier()
        pltpu.sync_copy(acc_sp.at[pl.ds(lo, NS)],
                        acc_out.at[pl.ds(c * NP + lo, NS)])
        pltpu.sync_copy(t_sp.at[pl.ds(lo, NS)],
                        t_out.at[pl.ds(c * NP + lo, NS)])

    return k(src2, dst2, g0, dinv, z1, z2)


def _tc_final(acc2, t2, dinv, g0, b1, W2, b2, N, NP, BN):
    H = g0.shape[1]
    M = W2.shape[1]
    nblk = N // BN

    def body(a_ref, t_ref, dv_ref, g0_ref, b1_ref, w2_ref, b2_ref,
             o_ref, pool_ref):
        i = pl.program_id(0)
        accs = a_ref[0] + a_ref[1]
        ts = t_ref[0] + t_ref[1]
        dv = dv_ref[...]
        h1 = jnp.maximum(dv * (accs + g0_ref[...]) + b1_ref[...], 0.0)
        cvec = dv * (ts + dv)
        part = jnp.sum(cvec * h1, axis=0, keepdims=True)

        @pl.when(i == 0)
        def _():
            pool_ref[...] = jnp.zeros_like(pool_ref)

        pool_ref[...] += part

        @pl.when(i == nblk - 1)
        def _():
            o_ref[...] = jnp.dot(pool_ref[...] * (1.0 / N), w2_ref[...],
                                 preferred_element_type=jnp.float32) \
                         + b2_ref[...]

    return pl.pallas_call(
        body,
        grid=(nblk,),
        in_specs=[pl.BlockSpec((2, BN, H), lambda i: (0, i, 0)),
                  pl.BlockSpec((2, BN, 1), lambda i: (0, i, 0)),
                  pl.BlockSpec((BN, 1), lambda i: (i, 0)),
                  pl.BlockSpec((BN, H), lambda i: (i, 0)),
                  pl.BlockSpec((1, H), lambda i: (0, 0)),
                  pl.BlockSpec((H, M), lambda i: (0, 0)),
                  pl.BlockSpec((1, M), lambda i: (0, 0))],
        out_specs=pl.BlockSpec((1, M), lambda i: (0, 0)),
        out_shape=jax.ShapeDtypeStruct((1, M), jnp.float32),
        scratch_shapes=[pltpu.VMEM((1, H), jnp.float32)],
    )(acc2.reshape(2, NP, H), t2.reshape(2, NP, 1), dinv, g0,
      b1.reshape(1, H), W2, b2.reshape(1, M))


def kernel(x, edge_index, W1, b1, W2, b2):
    N, F = x.shape
    H = W1.shape[1]
    E = edge_index.shape[1]

    NP = ((N + 127) // 128) * 128
    R = -(-E // (NW * B))
    R = ((R + CCH - 1) // CCH) * CCH
    EP = NW * R * B
    pad = EP - E

    padidx = N + (jnp.arange(pad, dtype=jnp.int32) % 16)
    src2 = jnp.concatenate([edge_index[0], padidx]).reshape(NW * R, B)
    dst2 = jnp.concatenate([edge_index[1], padidx]).reshape(NW * R, B)

    z1 = jnp.zeros((NP,), jnp.float32)
    z2 = jnp.zeros((NP, H), jnp.float32)
    ones = jnp.ones((B,), jnp.float32)

    h0 = _tc_h0(x, W1, NP, 1000)
    deg2 = _sc_deg(dst2, z1, ones, NP)
    dinv, g0 = _tc_dinv_g0(deg2, h0, N, NP, 1000)
    acc2, t2 = _sc_agg(src2, dst2, g0, dinv.reshape(NP), z1, z2, NP)
    return _tc_final(acc2, t2, dinv, g0, b1, W2, b2, N, NP, 1000)
